# baseline (speedup 1.0000x reference)
import functools

import jax
import jax.numpy as jnp
from jax import lax
from jax.experimental import pallas as pl
from jax.experimental.pallas import tpu as pltpu
from jax.experimental.pallas import tpu_sc as plsc

R = 128
N = 32768
K = 3
L = 16
NC = 2
NS = 16
NW = NC * NS
R_SC = 32
R_TC = R - R_SC
TCB = 8
ROWS_PER_W = R_SC // NW
NSETS = 4
SEG = 32
SEGW = SEG * L
NSEG = N // SEGW
G2 = 4
OUTW = 8

NEG = float("-inf")
IMAX = 2**31 - 1


def _scan_insert(m, i, v, iv):
    m1, m2, m3 = m
    i1, i2, i3 = i
    c1 = v > m1
    u1 = jnp.minimum(v, m1)
    nm1 = jnp.maximum(v, m1)
    ni1 = jnp.where(c1, iv, i1)
    iu1 = jnp.where(c1, i1, iv)
    c2 = u1 > m2
    u2 = jnp.minimum(u1, m2)
    nm2 = jnp.maximum(u1, m2)
    ni2 = jnp.where(c2, iu1, i2)
    iu2 = jnp.where(c2, i2, iu1)
    c3 = u2 > m3
    nm3 = jnp.maximum(u2, m3)
    ni3 = jnp.where(c3, iu2, i3)
    return (nm1, nm2, nm3), (ni1, ni2, ni3)


def _lex_insert(m, i, v, iv):
    m1, m2, m3 = m
    i1, i2, i3 = i
    c1 = (v > m1) | ((v == m1) & (iv < i1))
    nm1 = jnp.where(c1, v, m1)
    u1 = jnp.where(c1, m1, v)
    ni1 = jnp.where(c1, iv, i1)
    iu1 = jnp.where(c1, i1, iv)
    c2 = (u1 > m2) | ((u1 == m2) & (iu1 < i2))
    nm2 = jnp.where(c2, u1, m2)
    u2 = jnp.where(c2, m2, u1)
    ni2 = jnp.where(c2, iu1, i2)
    iu2 = jnp.where(c2, i2, iu1)
    c3 = (u2 > m3) | ((u2 == m3) & (iu2 < i3))
    nm3 = jnp.where(c3, u2, m3)
    ni3 = jnp.where(c3, iu2, i3)
    return (nm1, nm2, nm3), (ni1, ni2, ni3)


def _row_topk(load_chunk, seg_store, seg_load, lane,
              fori=lax.fori_loop, cond=lax.cond):
    zf = lane * 0.0

    def pass_a(s, rowmax):
        base = s * SEGW
        vs = [load_chunk(base + u * L) for u in range(SEG)]
        while len(vs) > 1:
            vs = [jnp.maximum(a, b) for a, b in zip(vs[::2], vs[1::2])]
        seg_store(s, vs[0])
        return jnp.maximum(rowmax, vs[0])

    rowmax = fori(0, NSEG, pass_a, zf + NEG)

    m = rowmax
    for _ in range(K - 1):
        mx = jnp.max(m)
        elig = m == mx
        wl = jnp.min(jnp.where(elig, lane, L))
        m = jnp.where(lane == wl, NEG, m)
    tv = zf + jnp.max(m)

    init = (
        tuple((zf + NEG,) * 3 for _ in range(NSETS)),
        tuple((lane * 0 + IMAX,) * 3 for _ in range(NSETS)),
    )

    def seg_process(s, c2):
        segmax = seg_load(s)
        t2 = jnp.any(segmax >= tv)

        def do2(c3):
            def chunk_body(t, c4):
                ms, is_ = list(c4[0]), list(c4[1])
                base = s * SEGW + t * (NSETS * L)
                ivb = lane + base
                for j in range(NSETS):
                    v = load_chunk(base + j * L)
                    ms[j], is_[j] = _scan_insert(ms[j], is_[j], v, ivb + j * L)
                return tuple(ms), tuple(is_)

            return fori(0, SEG // NSETS, chunk_body, c3)

        return cond(t2, do2, lambda c3: c3, c2)

    def pass_b(g, carry):
        s0 = g * G2
        vs = [seg_load(s0 + j) for j in range(G2)]
        while len(vs) > 1:
            vs = [jnp.maximum(a, b) for a, b in zip(vs[::2], vs[1::2])]
        trig = jnp.any(vs[0] >= tv)
        return cond(trig, lambda c: fori(s0, s0 + G2, seg_process, c),
                    lambda c: c, carry)

    ms, is_ = fori(0, NSEG // G2, pass_b, init)

    m0, i0 = ms[0], is_[0]
    for s in range(1, NSETS):
        for j in range(3):
            m0, i0 = _lex_insert(m0, i0, ms[s][j], is_[s][j])

    m1, m2, m3 = m0
    i1, i2, i3 = i0
    rv = zf + NEG
    ri = lane * 0
    for k in range(K):
        mx = jnp.max(m1)
        elig = m1 == mx
        ix = jnp.min(jnp.where(elig, i1, IMAX))
        rv = jnp.where(lane == k, mx, rv)
        ri = jnp.where(lane == k, ix, ri)
        win = elig & (i1 == ix)
        m1 = jnp.where(win, m2, m1)
        i1 = jnp.where(win, i2, i1)
        m2 = jnp.where(win, m3, m2)
        i2 = jnp.where(win, i3, i2)
        m3 = jnp.where(win, NEG, m3)
        i3 = jnp.where(win, IMAX, i3)
    return rv, ri


@functools.cache
def _make_topk():
    mesh = plsc.VectorSubcoreMesh(
        core_axis_name="c", subcore_axis_name="s", num_cores=NC, num_subcores=NS
    )

    @functools.partial(
        pl.kernel,
        out_type=(
            jax.ShapeDtypeStruct((R_SC * OUTW,), jnp.float32),
            jax.ShapeDtypeStruct((R_SC * OUTW,), jnp.int32),
        ),
        mesh=mesh,
        compiler_params=pltpu.CompilerParams(
            needs_layout_passes=False,
            skip_device_barrier=True,
            disable_bounds_checks=True,
        ),
        scratch_types=[
            pltpu.VMEM((2 * N,), jnp.float32),
            pltpu.VMEM((NSEG * L,), jnp.float32),
            pltpu.VMEM((ROWS_PER_W * OUTW + L,), jnp.float32),
            pltpu.VMEM((ROWS_PER_W * OUTW + L,), jnp.int32),
            pltpu.SemaphoreType.DMA,
        ],
    )
    def k(x_hbm, outv_hbm, outi_hbm, buf, segbuf, rvf, rif, sem):
        wid = lax.axis_index("s") * NC + lax.axis_index("c")
        lane = lax.iota(jnp.int32, L)
        base_row = wid * ROWS_PER_W
        pltpu.async_copy(x_hbm.at[base_row], buf.at[pl.ds(0, N)], sem)

        def row_body(r, carry):
            boff = (r & 1) * N
            pltpu.make_async_copy(
                x_hbm.at[base_row + r], buf.at[pl.ds(boff, N)], sem
            ).wait()

            @pl.when(r < ROWS_PER_W - 1)
            def _prefetch():
                pltpu.async_copy(
                    x_hbm.at[base_row + r + 1], buf.at[pl.ds(N - boff, N)], sem
                )

            rv, ri = _row_topk(
                lambda off: buf[pl.ds(boff + off, L)],
                lambda s, v: segbuf.__setitem__(pl.ds(s * L, L), v),
                lambda s: segbuf[pl.ds(s * L, L)],
                lane,
            )
            msk = lane < OUTW
            plsc.store_compressed(rvf.at[pl.ds(r * OUTW, L)], rv, mask=msk)
            plsc.store_compressed(rif.at[pl.ds(r * OUTW, L)], ri, mask=msk)
            return carry

        lax.fori_loop(0, ROWS_PER_W, row_body, 0)
        nout = ROWS_PER_W * OUTW
        pltpu.sync_copy(rvf.at[pl.ds(0, nout)], outv_hbm.at[pl.ds(base_row * OUTW, nout)])
        pltpu.sync_copy(rif.at[pl.ds(0, nout)], outi_hbm.at[pl.ds(base_row * OUTW, nout)])

    return k


def _tc_body(x_ref, v_ref, i_ref):
    x = x_ref[...]
    iota = lax.broadcasted_iota(jnp.int32, (TCB, N), 1)
    vs, is_ = [], []
    for _ in range(K):
        m = jnp.max(x, axis=1, keepdims=True)
        idx = jnp.min(jnp.where(x == m, iota, IMAX), axis=1, keepdims=True)
        x = jnp.where(iota == idx, NEG, x)
        vs.append(m)
        is_.append(idx)
    v_ref[...] = jnp.concatenate(vs, axis=1)
    i_ref[...] = jnp.concatenate(is_, axis=1)


@functools.cache
def _make_tc_topk():
    return pl.pallas_call(
        _tc_body,
        grid=(R_TC // TCB,),
        in_specs=[pl.BlockSpec((TCB, N), lambda i: (i + R_SC // TCB, 0))],
        out_specs=[
            pl.BlockSpec((TCB, K), lambda i: (i, 0)),
            pl.BlockSpec((TCB, K), lambda i: (i, 0)),
        ],
        out_shape=(
            jax.ShapeDtypeStruct((R_TC, K), jnp.float32),
            jax.ShapeDtypeStruct((R_TC, K), jnp.int32),
        ),
    )


def kernel(x):
    sc_v, sc_i = _make_topk()(x)
    tc_v, tc_i = _make_tc_topk()(x)
    vals = jnp.concatenate([sc_v.reshape(R_SC, OUTW)[:, :K], tc_v], axis=0)
    idxs = jnp.concatenate([sc_i.reshape(R_SC, OUTW)[:, :K], tc_i], axis=0)
    return vals, idxs


# speedup vs baseline: 1.0005x; 1.0005x over previous
import functools

import jax
import jax.numpy as jnp
from jax import lax
from jax.experimental import pallas as pl
from jax.experimental.pallas import tpu as pltpu
from jax.experimental.pallas import tpu_sc as plsc

R = 128
N = 32768
K = 3
L = 16
NC = 2
NS = 16
NW = NC * NS
R_SC = 32
R_TC = R - R_SC
TCB = 8
ROWS_PER_W = R_SC // NW
NSETS = 4
SEG = 32
SEGW = SEG * L
NSEG = N // SEGW
G2 = 4
OUTW = 8

NEG = float("-inf")
IMAX = 2**31 - 1


def _scan_insert(m, i, v, iv):
    m1, m2, m3 = m
    i1, i2, i3 = i
    c1 = v > m1
    u1 = jnp.minimum(v, m1)
    nm1 = jnp.maximum(v, m1)
    ni1 = jnp.where(c1, iv, i1)
    iu1 = jnp.where(c1, i1, iv)
    c2 = u1 > m2
    u2 = jnp.minimum(u1, m2)
    nm2 = jnp.maximum(u1, m2)
    ni2 = jnp.where(c2, iu1, i2)
    iu2 = jnp.where(c2, i2, iu1)
    c3 = u2 > m3
    nm3 = jnp.maximum(u2, m3)
    ni3 = jnp.where(c3, iu2, i3)
    return (nm1, nm2, nm3), (ni1, ni2, ni3)


def _lex_insert(m, i, v, iv):
    m1, m2, m3 = m
    i1, i2, i3 = i
    c1 = (v > m1) | ((v == m1) & (iv < i1))
    nm1 = jnp.where(c1, v, m1)
    u1 = jnp.where(c1, m1, v)
    ni1 = jnp.where(c1, iv, i1)
    iu1 = jnp.where(c1, i1, iv)
    c2 = (u1 > m2) | ((u1 == m2) & (iu1 < i2))
    nm2 = jnp.where(c2, u1, m2)
    u2 = jnp.where(c2, m2, u1)
    ni2 = jnp.where(c2, iu1, i2)
    iu2 = jnp.where(c2, i2, iu1)
    c3 = (u2 > m3) | ((u2 == m3) & (iu2 < i3))
    nm3 = jnp.where(c3, u2, m3)
    ni3 = jnp.where(c3, iu2, i3)
    return (nm1, nm2, nm3), (ni1, ni2, ni3)


def _row_topk(load_chunk, seg_store, seg_load, lane,
              fori=lax.fori_loop, cond=lax.cond):
    zf = lane * 0.0

    def pass_a(s, rowmax):
        base = s * SEGW
        vs = [load_chunk(base + u * L) for u in range(SEG)]
        while len(vs) > 1:
            vs = [jnp.maximum(a, b) for a, b in zip(vs[::2], vs[1::2])]
        seg_store(s, vs[0])
        return jnp.maximum(rowmax, vs[0])

    rowmax = fori(0, NSEG, pass_a, zf + NEG)

    m = rowmax
    for _ in range(K - 1):
        mx = jnp.max(m)
        elig = m == mx
        wl = jnp.min(jnp.where(elig, lane, L))
        m = jnp.where(lane == wl, NEG, m)
    tv = zf + jnp.max(m)

    init = (
        tuple((zf + NEG,) * 3 for _ in range(NSETS)),
        tuple((lane * 0 + IMAX,) * 3 for _ in range(NSETS)),
    )

    def seg_process(s, c2):
        segmax = seg_load(s)
        t2 = jnp.any(segmax >= tv)

        def do2(c3):
            def chunk_body(t, c4):
                ms, is_ = list(c4[0]), list(c4[1])
                base = s * SEGW + t * (NSETS * L)
                ivb = lane + base
                for j in range(NSETS):
                    v = load_chunk(base + j * L)
                    ms[j], is_[j] = _scan_insert(ms[j], is_[j], v, ivb + j * L)
                return tuple(ms), tuple(is_)

            return fori(0, SEG // NSETS, chunk_body, c3)

        return cond(t2, do2, lambda c3: c3, c2)

    def pass_b(g, carry):
        s0 = g * G2
        vs = [seg_load(s0 + j) for j in range(G2)]
        while len(vs) > 1:
            vs = [jnp.maximum(a, b) for a, b in zip(vs[::2], vs[1::2])]
        trig = jnp.any(vs[0] >= tv)
        return cond(trig, lambda c: fori(s0, s0 + G2, seg_process, c),
                    lambda c: c, carry)

    ms, is_ = fori(0, NSEG // G2, pass_b, init)

    m0, i0 = ms[0], is_[0]
    for s in range(1, NSETS):
        for j in range(3):
            m0, i0 = _lex_insert(m0, i0, ms[s][j], is_[s][j])

    m1, m2, m3 = m0
    i1, i2, i3 = i0
    rv = zf + NEG
    ri = lane * 0
    for k in range(K):
        mx = jnp.max(m1)
        elig = m1 == mx
        ix = jnp.min(jnp.where(elig, i1, IMAX))
        rv = jnp.where(lane == k, mx, rv)
        ri = jnp.where(lane == k, ix, ri)
        win = elig & (i1 == ix)
        m1 = jnp.where(win, m2, m1)
        i1 = jnp.where(win, i2, i1)
        m2 = jnp.where(win, m3, m2)
        i2 = jnp.where(win, i3, i2)
        m3 = jnp.where(win, NEG, m3)
        i3 = jnp.where(win, IMAX, i3)
    return rv, ri


@functools.cache
def _make_topk():
    mesh = plsc.VectorSubcoreMesh(
        core_axis_name="c", subcore_axis_name="s", num_cores=NC, num_subcores=NS
    )

    @functools.partial(
        pl.kernel,
        out_type=(
            jax.ShapeDtypeStruct((R_SC * OUTW,), jnp.float32),
            jax.ShapeDtypeStruct((R_SC * OUTW,), jnp.int32),
        ),
        mesh=mesh,
        compiler_params=pltpu.CompilerParams(
            needs_layout_passes=False,
            skip_device_barrier=True,
            disable_bounds_checks=True,
        ),
        scratch_types=[
            pltpu.VMEM((2 * N,), jnp.float32),
            pltpu.VMEM((NSEG * L,), jnp.float32),
            pltpu.VMEM((ROWS_PER_W * OUTW + L,), jnp.float32),
            pltpu.VMEM((ROWS_PER_W * OUTW + L,), jnp.int32),
            pltpu.SemaphoreType.DMA,
        ],
    )
    def k(x_hbm, outv_hbm, outi_hbm, buf, segbuf, rvf, rif, sem):
        wid = lax.axis_index("s") * NC + lax.axis_index("c")
        lane = lax.iota(jnp.int32, L)
        base_row = wid * ROWS_PER_W
        pltpu.async_copy(x_hbm.at[base_row], buf.at[pl.ds(0, N)], sem)

        def row_body(r, carry):
            boff = (r & 1) * N
            pltpu.make_async_copy(
                x_hbm.at[base_row + r], buf.at[pl.ds(boff, N)], sem
            ).wait()

            @pl.when(r < ROWS_PER_W - 1)
            def _prefetch():
                pltpu.async_copy(
                    x_hbm.at[base_row + r + 1], buf.at[pl.ds(N - boff, N)], sem
                )

            rv, ri = _row_topk(
                lambda off: buf[pl.ds(boff + off, L)],
                lambda s, v: segbuf.__setitem__(pl.ds(s * L, L), v),
                lambda s: segbuf[pl.ds(s * L, L)],
                lane,
            )
            msk = lane < OUTW
            plsc.store_compressed(rvf.at[pl.ds(r * OUTW, L)], rv, mask=msk)
            plsc.store_compressed(rif.at[pl.ds(r * OUTW, L)], ri, mask=msk)
            return carry

        lax.fori_loop(0, ROWS_PER_W, row_body, 0)
        nout = ROWS_PER_W * OUTW
        pltpu.sync_copy(rvf.at[pl.ds(0, nout)], outv_hbm.at[pl.ds(base_row * OUTW, nout)])
        pltpu.sync_copy(rif.at[pl.ds(0, nout)], outi_hbm.at[pl.ds(base_row * OUTW, nout)])

    return k


def _tc_body(x_ref, v_ref, i_ref):
    x = x_ref[...]
    iota = lax.broadcasted_iota(jnp.int32, (TCB, N), 1)
    vs, is_ = [], []
    for _ in range(K):
        m = jnp.max(x, axis=1, keepdims=True)
        idx = jnp.min(jnp.where(x == m, iota, IMAX), axis=1, keepdims=True)
        x = jnp.where(iota == idx, NEG, x)
        vs.append(m)
        is_.append(idx)
    v_ref[...] = jnp.concatenate(vs, axis=1)
    i_ref[...] = jnp.concatenate(is_, axis=1)


@functools.cache
def _make_tc_topk():
    return pl.pallas_call(
        _tc_body,
        grid=(R_TC // TCB,),
        in_specs=[pl.BlockSpec((TCB, N), lambda i: (i + R_SC // TCB, 0))],
        out_specs=[
            pl.BlockSpec((TCB, K), lambda i: (i, 0)),
            pl.BlockSpec((TCB, K), lambda i: (i, 0)),
        ],
        out_shape=(
            jax.ShapeDtypeStruct((R_TC, K), jnp.float32),
            jax.ShapeDtypeStruct((R_TC, K), jnp.int32),
        ),
    )


def kernel(x):
    tc_v, tc_i = _make_tc_topk()(x)
    sc_v, sc_i = _make_topk()(x)
    vals = jnp.concatenate([sc_v.reshape(R_SC, OUTW)[:, :K], tc_v], axis=0)
    idxs = jnp.concatenate([sc_i.reshape(R_SC, OUTW)[:, :K], tc_i], axis=0)
    return vals, idxs


# speedup vs baseline: 1.0314x; 1.0310x over previous
import functools

import jax
import jax.numpy as jnp
from jax import lax
from jax.experimental import pallas as pl
from jax.experimental.pallas import tpu as pltpu
from jax.experimental.pallas import tpu_sc as plsc

R = 128
N = 32768
K = 3
L = 16
NC = 1
NS = 16
NW = NC * NS
R_SC = 32
R_TC = R - R_SC
TCB = 8
ROWS_PER_W = R_SC // NW
NSETS = 4
SEG = 32
SEGW = SEG * L
NSEG = N // SEGW
G2 = 4
OUTW = 8

NEG = float("-inf")
IMAX = 2**31 - 1


def _scan_insert(m, i, v, iv):
    m1, m2, m3 = m
    i1, i2, i3 = i
    c1 = v > m1
    u1 = jnp.minimum(v, m1)
    nm1 = jnp.maximum(v, m1)
    ni1 = jnp.where(c1, iv, i1)
    iu1 = jnp.where(c1, i1, iv)
    c2 = u1 > m2
    u2 = jnp.minimum(u1, m2)
    nm2 = jnp.maximum(u1, m2)
    ni2 = jnp.where(c2, iu1, i2)
    iu2 = jnp.where(c2, i2, iu1)
    c3 = u2 > m3
    nm3 = jnp.maximum(u2, m3)
    ni3 = jnp.where(c3, iu2, i3)
    return (nm1, nm2, nm3), (ni1, ni2, ni3)


def _lex_insert(m, i, v, iv):
    m1, m2, m3 = m
    i1, i2, i3 = i
    c1 = (v > m1) | ((v == m1) & (iv < i1))
    nm1 = jnp.where(c1, v, m1)
    u1 = jnp.where(c1, m1, v)
    ni1 = jnp.where(c1, iv, i1)
    iu1 = jnp.where(c1, i1, iv)
    c2 = (u1 > m2) | ((u1 == m2) & (iu1 < i2))
    nm2 = jnp.where(c2, u1, m2)
    u2 = jnp.where(c2, m2, u1)
    ni2 = jnp.where(c2, iu1, i2)
    iu2 = jnp.where(c2, i2, iu1)
    c3 = (u2 > m3) | ((u2 == m3) & (iu2 < i3))
    nm3 = jnp.where(c3, u2, m3)
    ni3 = jnp.where(c3, iu2, i3)
    return (nm1, nm2, nm3), (ni1, ni2, ni3)


def _row_topk(load_chunk, seg_store, seg_load, lane,
              fori=lax.fori_loop, cond=lax.cond):
    zf = lane * 0.0

    def pass_a(s, rowmax):
        base = s * SEGW
        vs = [load_chunk(base + u * L) for u in range(SEG)]
        while len(vs) > 1:
            vs = [jnp.maximum(a, b) for a, b in zip(vs[::2], vs[1::2])]
        seg_store(s, vs[0])
        return jnp.maximum(rowmax, vs[0])

    rowmax = fori(0, NSEG, pass_a, zf + NEG)

    m = rowmax
    for _ in range(K - 1):
        mx = jnp.max(m)
        elig = m == mx
        wl = jnp.min(jnp.where(elig, lane, L))
        m = jnp.where(lane == wl, NEG, m)
    tv = zf + jnp.max(m)

    init = (
        tuple((zf + NEG,) * 3 for _ in range(NSETS)),
        tuple((lane * 0 + IMAX,) * 3 for _ in range(NSETS)),
    )

    def seg_process(s, c2):
        segmax = seg_load(s)
        t2 = jnp.any(segmax >= tv)

        def do2(c3):
            def chunk_body(t, c4):
                ms, is_ = list(c4[0]), list(c4[1])
                base = s * SEGW + t * (NSETS * L)
                ivb = lane + base
                for j in range(NSETS):
                    v = load_chunk(base + j * L)
                    ms[j], is_[j] = _scan_insert(ms[j], is_[j], v, ivb + j * L)
                return tuple(ms), tuple(is_)

            return fori(0, SEG // NSETS, chunk_body, c3)

        return cond(t2, do2, lambda c3: c3, c2)

    def pass_b(g, carry):
        s0 = g * G2
        vs = [seg_load(s0 + j) for j in range(G2)]
        while len(vs) > 1:
            vs = [jnp.maximum(a, b) for a, b in zip(vs[::2], vs[1::2])]
        trig = jnp.any(vs[0] >= tv)
        return cond(trig, lambda c: fori(s0, s0 + G2, seg_process, c),
                    lambda c: c, carry)

    ms, is_ = fori(0, NSEG // G2, pass_b, init)

    m0, i0 = ms[0], is_[0]
    for s in range(1, NSETS):
        for j in range(3):
            m0, i0 = _lex_insert(m0, i0, ms[s][j], is_[s][j])

    m1, m2, m3 = m0
    i1, i2, i3 = i0
    rv = zf + NEG
    ri = lane * 0
    for k in range(K):
        mx = jnp.max(m1)
        elig = m1 == mx
        ix = jnp.min(jnp.where(elig, i1, IMAX))
        rv = jnp.where(lane == k, mx, rv)
        ri = jnp.where(lane == k, ix, ri)
        win = elig & (i1 == ix)
        m1 = jnp.where(win, m2, m1)
        i1 = jnp.where(win, i2, i1)
        m2 = jnp.where(win, m3, m2)
        i2 = jnp.where(win, i3, i2)
        m3 = jnp.where(win, NEG, m3)
        i3 = jnp.where(win, IMAX, i3)
    return rv, ri


@functools.cache
def _make_topk():
    mesh = plsc.VectorSubcoreMesh(
        core_axis_name="c", subcore_axis_name="s", num_cores=NC, num_subcores=NS
    )

    @functools.partial(
        pl.kernel,
        out_type=(
            jax.ShapeDtypeStruct((R_SC * OUTW,), jnp.float32),
            jax.ShapeDtypeStruct((R_SC * OUTW,), jnp.int32),
        ),
        mesh=mesh,
        compiler_params=pltpu.CompilerParams(
            needs_layout_passes=False,
            skip_device_barrier=True,
            disable_bounds_checks=True,
        ),
        scratch_types=[
            pltpu.VMEM((2 * N,), jnp.float32),
            pltpu.VMEM((NSEG * L,), jnp.float32),
            pltpu.VMEM((ROWS_PER_W * OUTW + L,), jnp.float32),
            pltpu.VMEM((ROWS_PER_W * OUTW + L,), jnp.int32),
            pltpu.SemaphoreType.DMA,
        ],
    )
    def k(x_hbm, outv_hbm, outi_hbm, buf, segbuf, rvf, rif, sem):
        wid = lax.axis_index("s") * NC + lax.axis_index("c")
        lane = lax.iota(jnp.int32, L)
        base_row = wid * ROWS_PER_W
        pltpu.async_copy(x_hbm.at[base_row], buf.at[pl.ds(0, N)], sem)

        def row_body(r, carry):
            boff = (r & 1) * N
            pltpu.make_async_copy(
                x_hbm.at[base_row + r], buf.at[pl.ds(boff, N)], sem
            ).wait()

            @pl.when(r < ROWS_PER_W - 1)
            def _prefetch():
                pltpu.async_copy(
                    x_hbm.at[base_row + r + 1], buf.at[pl.ds(N - boff, N)], sem
                )

            rv, ri = _row_topk(
                lambda off: buf[pl.ds(boff + off, L)],
                lambda s, v: segbuf.__setitem__(pl.ds(s * L, L), v),
                lambda s: segbuf[pl.ds(s * L, L)],
                lane,
            )
            msk = lane < OUTW
            plsc.store_compressed(rvf.at[pl.ds(r * OUTW, L)], rv, mask=msk)
            plsc.store_compressed(rif.at[pl.ds(r * OUTW, L)], ri, mask=msk)
            return carry

        lax.fori_loop(0, ROWS_PER_W, row_body, 0)
        nout = ROWS_PER_W * OUTW
        pltpu.sync_copy(rvf.at[pl.ds(0, nout)], outv_hbm.at[pl.ds(base_row * OUTW, nout)])
        pltpu.sync_copy(rif.at[pl.ds(0, nout)], outi_hbm.at[pl.ds(base_row * OUTW, nout)])

    return k


def _tc_body(x_ref, v_ref, i_ref):
    x = x_ref[...]
    iota = lax.broadcasted_iota(jnp.int32, (TCB, N), 1)
    vs, is_ = [], []
    for _ in range(K):
        m = jnp.max(x, axis=1, keepdims=True)
        idx = jnp.min(jnp.where(x == m, iota, IMAX), axis=1, keepdims=True)
        x = jnp.where(iota == idx, NEG, x)
        vs.append(m)
        is_.append(idx)
    v_ref[...] = jnp.concatenate(vs, axis=1)
    i_ref[...] = jnp.concatenate(is_, axis=1)


@functools.cache
def _make_tc_topk():
    return pl.pallas_call(
        _tc_body,
        grid=(R_TC // TCB,),
        in_specs=[pl.BlockSpec((TCB, N), lambda i: (i + R_SC // TCB, 0))],
        out_specs=[
            pl.BlockSpec((TCB, K), lambda i: (i, 0)),
            pl.BlockSpec((TCB, K), lambda i: (i, 0)),
        ],
        out_shape=(
            jax.ShapeDtypeStruct((R_TC, K), jnp.float32),
            jax.ShapeDtypeStruct((R_TC, K), jnp.int32),
        ),
    )


def kernel(x):
    tc_v, tc_i = _make_tc_topk()(x)
    sc_v, sc_i = _make_topk()(x)
    vals = jnp.concatenate([sc_v.reshape(R_SC, OUTW)[:, :K], tc_v], axis=0)
    idxs = jnp.concatenate([sc_i.reshape(R_SC, OUTW)[:, :K], tc_i], axis=0)
    return vals, idxs


# speedup vs baseline: 1.3611x; 1.3197x over previous
import functools

import jax
import jax.numpy as jnp
from jax import lax
from jax.experimental import pallas as pl
from jax.experimental.pallas import tpu as pltpu
from jax.experimental.pallas import tpu_sc as plsc

R = 128
N = 32768
K = 3
L = 16
NC = 2
NS = 16
NW = NC * NS
R_SC = 96
R_TC = R - R_SC
TCB = 16
ROWS_PER_W = R_SC // NW
NSETS = 4
SEG = 32
SEGW = SEG * L
NSEG = N // SEGW
G2 = 4
OUTW = 8

NEG = float("-inf")
IMAX = 2**31 - 1


def _scan_insert(m, i, v, iv):
    m1, m2, m3 = m
    i1, i2, i3 = i
    c1 = v > m1
    u1 = jnp.minimum(v, m1)
    nm1 = jnp.maximum(v, m1)
    ni1 = jnp.where(c1, iv, i1)
    iu1 = jnp.where(c1, i1, iv)
    c2 = u1 > m2
    u2 = jnp.minimum(u1, m2)
    nm2 = jnp.maximum(u1, m2)
    ni2 = jnp.where(c2, iu1, i2)
    iu2 = jnp.where(c2, i2, iu1)
    c3 = u2 > m3
    nm3 = jnp.maximum(u2, m3)
    ni3 = jnp.where(c3, iu2, i3)
    return (nm1, nm2, nm3), (ni1, ni2, ni3)


def _lex_insert(m, i, v, iv):
    m1, m2, m3 = m
    i1, i2, i3 = i
    c1 = (v > m1) | ((v == m1) & (iv < i1))
    nm1 = jnp.where(c1, v, m1)
    u1 = jnp.where(c1, m1, v)
    ni1 = jnp.where(c1, iv, i1)
    iu1 = jnp.where(c1, i1, iv)
    c2 = (u1 > m2) | ((u1 == m2) & (iu1 < i2))
    nm2 = jnp.where(c2, u1, m2)
    u2 = jnp.where(c2, m2, u1)
    ni2 = jnp.where(c2, iu1, i2)
    iu2 = jnp.where(c2, i2, iu1)
    c3 = (u2 > m3) | ((u2 == m3) & (iu2 < i3))
    nm3 = jnp.where(c3, u2, m3)
    ni3 = jnp.where(c3, iu2, i3)
    return (nm1, nm2, nm3), (ni1, ni2, ni3)


def _row_topk(load_chunk, seg_store, seg_load, lane,
              fori=lax.fori_loop, cond=lax.cond):
    zf = lane * 0.0

    def pass_a(s, rowmax):
        base = s * SEGW
        vs = [load_chunk(base + u * L) for u in range(SEG)]
        while len(vs) > 1:
            vs = [jnp.maximum(a, b) for a, b in zip(vs[::2], vs[1::2])]
        seg_store(s, vs[0])
        return jnp.maximum(rowmax, vs[0])

    rowmax = fori(0, NSEG, pass_a, zf + NEG)

    m = rowmax
    for _ in range(K - 1):
        mx = jnp.max(m)
        elig = m == mx
        wl = jnp.min(jnp.where(elig, lane, L))
        m = jnp.where(lane == wl, NEG, m)
    tv = zf + jnp.max(m)

    init = (
        tuple((zf + NEG,) * 3 for _ in range(NSETS)),
        tuple((lane * 0 + IMAX,) * 3 for _ in range(NSETS)),
    )

    def seg_process(s, c2):
        segmax = seg_load(s)
        t2 = jnp.any(segmax >= tv)

        def do2(c3):
            def chunk_body(t, c4):
                ms, is_ = list(c4[0]), list(c4[1])
                base = s * SEGW + t * (NSETS * L)
                ivb = lane + base
                for j in range(NSETS):
                    v = load_chunk(base + j * L)
                    ms[j], is_[j] = _scan_insert(ms[j], is_[j], v, ivb + j * L)
                return tuple(ms), tuple(is_)

            return fori(0, SEG // NSETS, chunk_body, c3)

        return cond(t2, do2, lambda c3: c3, c2)

    def pass_b(g, carry):
        s0 = g * G2
        vs = [seg_load(s0 + j) for j in range(G2)]
        while len(vs) > 1:
            vs = [jnp.maximum(a, b) for a, b in zip(vs[::2], vs[1::2])]
        trig = jnp.any(vs[0] >= tv)
        return cond(trig, lambda c: fori(s0, s0 + G2, seg_process, c),
                    lambda c: c, carry)

    ms, is_ = fori(0, NSEG // G2, pass_b, init)

    m0, i0 = ms[0], is_[0]
    for s in range(1, NSETS):
        for j in range(3):
            m0, i0 = _lex_insert(m0, i0, ms[s][j], is_[s][j])

    m1, m2, m3 = m0
    i1, i2, i3 = i0
    rv = zf + NEG
    ri = lane * 0
    for k in range(K):
        mx = jnp.max(m1)
        elig = m1 == mx
        ix = jnp.min(jnp.where(elig, i1, IMAX))
        rv = jnp.where(lane == k, mx, rv)
        ri = jnp.where(lane == k, ix, ri)
        win = elig & (i1 == ix)
        m1 = jnp.where(win, m2, m1)
        i1 = jnp.where(win, i2, i1)
        m2 = jnp.where(win, m3, m2)
        i2 = jnp.where(win, i3, i2)
        m3 = jnp.where(win, NEG, m3)
        i3 = jnp.where(win, IMAX, i3)
    return rv, ri


@functools.cache
def _make_topk():
    mesh = plsc.VectorSubcoreMesh(
        core_axis_name="c", subcore_axis_name="s", num_cores=NC, num_subcores=NS
    )

    @functools.partial(
        pl.kernel,
        out_type=(
            jax.ShapeDtypeStruct((R_SC * OUTW,), jnp.float32),
            jax.ShapeDtypeStruct((R_SC * OUTW,), jnp.int32),
        ),
        mesh=mesh,
        compiler_params=pltpu.CompilerParams(
            needs_layout_passes=False,
            skip_device_barrier=True,
            disable_bounds_checks=True,
        ),
        scratch_types=[
            pltpu.VMEM((2 * N,), jnp.float32),
            pltpu.VMEM((NSEG * L,), jnp.float32),
            pltpu.VMEM((ROWS_PER_W * OUTW + L,), jnp.float32),
            pltpu.VMEM((ROWS_PER_W * OUTW + L,), jnp.int32),
            pltpu.SemaphoreType.DMA,
        ],
    )
    def k(x_hbm, outv_hbm, outi_hbm, buf, segbuf, rvf, rif, sem):
        wid = lax.axis_index("s") * NC + lax.axis_index("c")
        lane = lax.iota(jnp.int32, L)
        base_row = wid * ROWS_PER_W
        pltpu.async_copy(x_hbm.at[base_row], buf.at[pl.ds(0, N)], sem)

        def row_body(r, carry):
            boff = (r & 1) * N
            pltpu.make_async_copy(
                x_hbm.at[base_row + r], buf.at[pl.ds(boff, N)], sem
            ).wait()

            @pl.when(r < ROWS_PER_W - 1)
            def _prefetch():
                pltpu.async_copy(
                    x_hbm.at[base_row + r + 1], buf.at[pl.ds(N - boff, N)], sem
                )

            rv, ri = _row_topk(
                lambda off: buf[pl.ds(boff + off, L)],
                lambda s, v: segbuf.__setitem__(pl.ds(s * L, L), v),
                lambda s: segbuf[pl.ds(s * L, L)],
                lane,
            )
            msk = lane < OUTW
            plsc.store_compressed(rvf.at[pl.ds(r * OUTW, L)], rv, mask=msk)
            plsc.store_compressed(rif.at[pl.ds(r * OUTW, L)], ri, mask=msk)
            return carry

        lax.fori_loop(0, ROWS_PER_W, row_body, 0)
        nout = ROWS_PER_W * OUTW
        pltpu.sync_copy(rvf.at[pl.ds(0, nout)], outv_hbm.at[pl.ds(base_row * OUTW, nout)])
        pltpu.sync_copy(rif.at[pl.ds(0, nout)], outi_hbm.at[pl.ds(base_row * OUTW, nout)])

    return k


def _tc_body(x_ref, v_ref, i_ref):
    x = x_ref[...]
    iota = lax.broadcasted_iota(jnp.int32, (TCB, N), 1)
    vs, is_ = [], []
    for _ in range(K):
        m = jnp.max(x, axis=1, keepdims=True)
        idx = jnp.min(jnp.where(x == m, iota, IMAX), axis=1, keepdims=True)
        x = jnp.where(iota == idx, NEG, x)
        vs.append(m)
        is_.append(idx)
    v_ref[...] = jnp.concatenate(vs, axis=1)
    i_ref[...] = jnp.concatenate(is_, axis=1)


@functools.cache
def _make_tc_topk():
    return pl.pallas_call(
        _tc_body,
        grid=(R_TC // TCB,),
        in_specs=[pl.BlockSpec((TCB, N), lambda i: (i + R_SC // TCB, 0))],
        out_specs=[
            pl.BlockSpec((TCB, K), lambda i: (i, 0)),
            pl.BlockSpec((TCB, K), lambda i: (i, 0)),
        ],
        out_shape=(
            jax.ShapeDtypeStruct((R_TC, K), jnp.float32),
            jax.ShapeDtypeStruct((R_TC, K), jnp.int32),
        ),
    )


def kernel(x):
    tc_v, tc_i = _make_tc_topk()(x)
    sc_v, sc_i = _make_topk()(x)
    vals = jnp.concatenate([sc_v.reshape(R_SC, OUTW)[:, :K], tc_v], axis=0)
    idxs = jnp.concatenate([sc_i.reshape(R_SC, OUTW)[:, :K], tc_i], axis=0)
    return vals, idxs


# speedup vs baseline: 1.4623x; 1.0743x over previous
import functools

import jax
import jax.numpy as jnp
from jax import lax
from jax.experimental import pallas as pl
from jax.experimental.pallas import tpu as pltpu
from jax.experimental.pallas import tpu_sc as plsc

R = 128
N = 32768
K = 3
L = 16
NC = 2
NS = 16
NW = NC * NS
R_SC = 64
R_TC = R - R_SC
TCB = 16
ROWS_PER_W = R_SC // NW
NSETS = 4
SEG = 32
SEGW = SEG * L
NSEG = N // SEGW
G2 = 4
OUTW = 8

NEG = float("-inf")
IMAX = 2**31 - 1


def _scan_insert(m, i, v, iv):
    m1, m2, m3 = m
    i1, i2, i3 = i
    c1 = v > m1
    u1 = jnp.minimum(v, m1)
    nm1 = jnp.maximum(v, m1)
    ni1 = jnp.where(c1, iv, i1)
    iu1 = jnp.where(c1, i1, iv)
    c2 = u1 > m2
    u2 = jnp.minimum(u1, m2)
    nm2 = jnp.maximum(u1, m2)
    ni2 = jnp.where(c2, iu1, i2)
    iu2 = jnp.where(c2, i2, iu1)
    c3 = u2 > m3
    nm3 = jnp.maximum(u2, m3)
    ni3 = jnp.where(c3, iu2, i3)
    return (nm1, nm2, nm3), (ni1, ni2, ni3)


def _lex_insert(m, i, v, iv):
    m1, m2, m3 = m
    i1, i2, i3 = i
    c1 = (v > m1) | ((v == m1) & (iv < i1))
    nm1 = jnp.where(c1, v, m1)
    u1 = jnp.where(c1, m1, v)
    ni1 = jnp.where(c1, iv, i1)
    iu1 = jnp.where(c1, i1, iv)
    c2 = (u1 > m2) | ((u1 == m2) & (iu1 < i2))
    nm2 = jnp.where(c2, u1, m2)
    u2 = jnp.where(c2, m2, u1)
    ni2 = jnp.where(c2, iu1, i2)
    iu2 = jnp.where(c2, i2, iu1)
    c3 = (u2 > m3) | ((u2 == m3) & (iu2 < i3))
    nm3 = jnp.where(c3, u2, m3)
    ni3 = jnp.where(c3, iu2, i3)
    return (nm1, nm2, nm3), (ni1, ni2, ni3)


def _row_topk(load_chunk, seg_store, seg_load, lane,
              fori=lax.fori_loop, cond=lax.cond):
    zf = lane * 0.0

    def pass_a(s, rowmax):
        base = s * SEGW
        vs = [load_chunk(base + u * L) for u in range(SEG)]
        while len(vs) > 1:
            vs = [jnp.maximum(a, b) for a, b in zip(vs[::2], vs[1::2])]
        seg_store(s, vs[0])
        return jnp.maximum(rowmax, vs[0])

    rowmax = fori(0, NSEG, pass_a, zf + NEG)

    m = rowmax
    for _ in range(K - 1):
        mx = jnp.max(m)
        elig = m == mx
        wl = jnp.min(jnp.where(elig, lane, L))
        m = jnp.where(lane == wl, NEG, m)
    tv = zf + jnp.max(m)

    init = (
        tuple((zf + NEG,) * 3 for _ in range(NSETS)),
        tuple((lane * 0 + IMAX,) * 3 for _ in range(NSETS)),
    )

    def seg_process(s, c2):
        segmax = seg_load(s)
        t2 = jnp.any(segmax >= tv)

        def do2(c3):
            def chunk_body(t, c4):
                ms, is_ = list(c4[0]), list(c4[1])
                base = s * SEGW + t * (NSETS * L)
                ivb = lane + base
                for j in range(NSETS):
                    v = load_chunk(base + j * L)
                    ms[j], is_[j] = _scan_insert(ms[j], is_[j], v, ivb + j * L)
                return tuple(ms), tuple(is_)

            return fori(0, SEG // NSETS, chunk_body, c3)

        return cond(t2, do2, lambda c3: c3, c2)

    def pass_b(g, carry):
        s0 = g * G2
        vs = [seg_load(s0 + j) for j in range(G2)]
        while len(vs) > 1:
            vs = [jnp.maximum(a, b) for a, b in zip(vs[::2], vs[1::2])]
        trig = jnp.any(vs[0] >= tv)
        return cond(trig, lambda c: fori(s0, s0 + G2, seg_process, c),
                    lambda c: c, carry)

    ms, is_ = fori(0, NSEG // G2, pass_b, init)

    m0, i0 = ms[0], is_[0]
    for s in range(1, NSETS):
        for j in range(3):
            m0, i0 = _lex_insert(m0, i0, ms[s][j], is_[s][j])

    m1, m2, m3 = m0
    i1, i2, i3 = i0
    rv = zf + NEG
    ri = lane * 0
    for k in range(K):
        mx = jnp.max(m1)
        elig = m1 == mx
        ix = jnp.min(jnp.where(elig, i1, IMAX))
        rv = jnp.where(lane == k, mx, rv)
        ri = jnp.where(lane == k, ix, ri)
        win = elig & (i1 == ix)
        m1 = jnp.where(win, m2, m1)
        i1 = jnp.where(win, i2, i1)
        m2 = jnp.where(win, m3, m2)
        i2 = jnp.where(win, i3, i2)
        m3 = jnp.where(win, NEG, m3)
        i3 = jnp.where(win, IMAX, i3)
    return rv, ri


@functools.cache
def _make_topk():
    mesh = plsc.VectorSubcoreMesh(
        core_axis_name="c", subcore_axis_name="s", num_cores=NC, num_subcores=NS
    )

    @functools.partial(
        pl.kernel,
        out_type=(
            jax.ShapeDtypeStruct((R_SC * OUTW,), jnp.float32),
            jax.ShapeDtypeStruct((R_SC * OUTW,), jnp.int32),
        ),
        mesh=mesh,
        compiler_params=pltpu.CompilerParams(
            needs_layout_passes=False,
            skip_device_barrier=True,
            disable_bounds_checks=True,
        ),
        scratch_types=[
            pltpu.VMEM((2 * N,), jnp.float32),
            pltpu.VMEM((NSEG * L,), jnp.float32),
            pltpu.VMEM((ROWS_PER_W * OUTW + L,), jnp.float32),
            pltpu.VMEM((ROWS_PER_W * OUTW + L,), jnp.int32),
            pltpu.SemaphoreType.DMA,
        ],
    )
    def k(x_hbm, outv_hbm, outi_hbm, buf, segbuf, rvf, rif, sem):
        wid = lax.axis_index("s") * NC + lax.axis_index("c")
        lane = lax.iota(jnp.int32, L)
        base_row = wid * ROWS_PER_W
        pltpu.async_copy(x_hbm.at[base_row], buf.at[pl.ds(0, N)], sem)

        def row_body(r, carry):
            boff = (r & 1) * N
            pltpu.make_async_copy(
                x_hbm.at[base_row + r], buf.at[pl.ds(boff, N)], sem
            ).wait()

            @pl.when(r < ROWS_PER_W - 1)
            def _prefetch():
                pltpu.async_copy(
                    x_hbm.at[base_row + r + 1], buf.at[pl.ds(N - boff, N)], sem
                )

            rv, ri = _row_topk(
                lambda off: buf[pl.ds(boff + off, L)],
                lambda s, v: segbuf.__setitem__(pl.ds(s * L, L), v),
                lambda s: segbuf[pl.ds(s * L, L)],
                lane,
            )
            msk = lane < OUTW
            plsc.store_compressed(rvf.at[pl.ds(r * OUTW, L)], rv, mask=msk)
            plsc.store_compressed(rif.at[pl.ds(r * OUTW, L)], ri, mask=msk)
            return carry

        lax.fori_loop(0, ROWS_PER_W, row_body, 0)
        nout = ROWS_PER_W * OUTW
        pltpu.sync_copy(rvf.at[pl.ds(0, nout)], outv_hbm.at[pl.ds(base_row * OUTW, nout)])
        pltpu.sync_copy(rif.at[pl.ds(0, nout)], outi_hbm.at[pl.ds(base_row * OUTW, nout)])

    return k


def _tc_body(x_ref, v_ref, i_ref):
    x = x_ref[...]
    iota = lax.broadcasted_iota(jnp.int32, (TCB, N), 1)
    vs, is_ = [], []
    for _ in range(K):
        m = jnp.max(x, axis=1, keepdims=True)
        idx = jnp.min(jnp.where(x == m, iota, IMAX), axis=1, keepdims=True)
        x = jnp.where(iota == idx, NEG, x)
        vs.append(m)
        is_.append(idx)
    v_ref[...] = jnp.concatenate(vs, axis=1)
    i_ref[...] = jnp.concatenate(is_, axis=1)


@functools.cache
def _make_tc_topk():
    return pl.pallas_call(
        _tc_body,
        grid=(R_TC // TCB,),
        in_specs=[pl.BlockSpec((TCB, N), lambda i: (i + R_SC // TCB, 0))],
        out_specs=[
            pl.BlockSpec((TCB, K), lambda i: (i, 0)),
            pl.BlockSpec((TCB, K), lambda i: (i, 0)),
        ],
        out_shape=(
            jax.ShapeDtypeStruct((R_TC, K), jnp.float32),
            jax.ShapeDtypeStruct((R_TC, K), jnp.int32),
        ),
    )


def kernel(x):
    tc_v, tc_i = _make_tc_topk()(x)
    sc_v, sc_i = _make_topk()(x)
    vals = jnp.concatenate([sc_v.reshape(R_SC, OUTW)[:, :K], tc_v], axis=0)
    idxs = jnp.concatenate([sc_i.reshape(R_SC, OUTW)[:, :K], tc_i], axis=0)
    return vals, idxs
